# baseline (speedup 1.0000x reference)
import functools

import jax
import jax.numpy as jnp
from jax import lax
from jax.experimental import pallas as pl
from jax.experimental.pallas import tpu as pltpu
from jax.experimental.pallas import tpu_sc as plsc

D = 1024
CHUNK = 16
NBUF = 4


@functools.cache
def _gather_fn(B):
    info = plsc.get_sparse_core_info()
    nw = info.num_cores * info.num_subcores
    b_per_w = B // nw
    nchunk = b_per_w // CHUNK
    nsuper = nchunk // NBUF
    mesh = plsc.VectorSubcoreMesh(core_axis_name="c", subcore_axis_name="s")

    @functools.partial(
        pl.kernel,
        mesh=mesh,
        out_type=jax.ShapeDtypeStruct((B, D), jnp.float32),
        scratch_types=[
            pltpu.VMEM((b_per_w,), jnp.int32),
            *[pltpu.VMEM((CHUNK, D), jnp.float32) for _ in range(NBUF)],
            *[pltpu.SemaphoreType.DMA for _ in range(2 * NBUF)],
        ],
    )
    def gather_kernel(table_hbm, ids_hbm, out_hbm, idx_v, *bufs_sems):
        bufs = bufs_sems[:NBUF]
        gsems = bufs_sems[NBUF : 2 * NBUF]
        osems = bufs_sems[2 * NBUF :]
        wid = lax.axis_index("s") * info.num_cores + lax.axis_index("c")
        base = wid * b_per_w

        pltpu.sync_copy(ids_hbm.at[pl.ds(base, b_per_w)], idx_v)

        def start_gather(g, b):
            pltpu.async_copy(
                table_hbm.at[idx_v.at[pl.ds(g * CHUNK, CHUNK)]], bufs[b], gsems[b]
            )

        for b in range(NBUF):
            start_gather(b, b)

        def super_body(k, carry):
            for b in range(NBUF):
                g = k * NBUF + b
                pltpu.make_async_copy(
                    table_hbm.at[idx_v.at[pl.ds(0, CHUNK)]], bufs[b], gsems[b]
                ).wait()
                pltpu.async_copy(
                    bufs[b],
                    out_hbm.at[pl.ds(base + g * CHUNK, CHUNK)],
                    osems[b],
                )
                @pl.when(g + NBUF < nchunk)
                def _():
                    pltpu.make_async_copy(
                        bufs[b],
                        out_hbm.at[pl.ds(base + g * CHUNK, CHUNK)],
                        osems[b],
                    ).wait()
                    start_gather(g + NBUF, b)

            return carry

        lax.fori_loop(0, nsuper, super_body, 0)

        for b in range(NBUF):
            pltpu.make_async_copy(
                bufs[b],
                out_hbm.at[pl.ds(base, CHUNK)],
                osems[b],
            ).wait()

    return gather_kernel


def kernel(input_ids, table):
    flat = input_ids.reshape(-1).astype(jnp.int32)
    out = _gather_fn(flat.shape[0])(table, flat)
    return out.reshape(input_ids.shape + (D,))


# speedup vs baseline: 1.0089x; 1.0089x over previous
import functools

import jax
import jax.numpy as jnp
from jax import lax
from jax.experimental import pallas as pl
from jax.experimental.pallas import tpu as pltpu
from jax.experimental.pallas import tpu_sc as plsc

D = 1024
CHUNK = 16
NBUF = 7


@functools.cache
def _gather_fn(B):
    info = plsc.get_sparse_core_info()
    nw = info.num_cores * info.num_subcores
    b_per_w = B // nw
    nchunk = b_per_w // CHUNK
    nsuper = -(-nchunk // NBUF)
    mesh = plsc.VectorSubcoreMesh(core_axis_name="c", subcore_axis_name="s")

    @functools.partial(
        pl.kernel,
        mesh=mesh,
        out_type=jax.ShapeDtypeStruct((B, D), jnp.float32),
        scratch_types=[
            pltpu.VMEM((b_per_w,), jnp.int32),
            *[pltpu.VMEM((CHUNK, D), jnp.float32) for _ in range(NBUF)],
            *[pltpu.SemaphoreType.DMA for _ in range(2 * NBUF)],
        ],
    )
    def gather_kernel(table_hbm, ids_hbm, out_hbm, idx_v, *bufs_sems):
        bufs = bufs_sems[:NBUF]
        gsems = bufs_sems[NBUF : 2 * NBUF]
        osems = bufs_sems[2 * NBUF :]
        wid = lax.axis_index("s") * info.num_cores + lax.axis_index("c")
        base = wid * b_per_w

        pltpu.sync_copy(ids_hbm.at[pl.ds(base, b_per_w)], idx_v)

        def start_gather(g, b):
            pltpu.async_copy(
                table_hbm.at[idx_v.at[pl.ds(g * CHUNK, CHUNK)]], bufs[b], gsems[b]
            )

        for b in range(NBUF):
            start_gather(b, b)

        def super_body(k, carry):
            for b in range(NBUF):
                g = k * NBUF + b

                @pl.when(g < nchunk)
                def _():
                    pltpu.make_async_copy(
                        table_hbm.at[idx_v.at[pl.ds(0, CHUNK)]], bufs[b], gsems[b]
                    ).wait()
                    pltpu.async_copy(
                        bufs[b],
                        out_hbm.at[pl.ds(base + g * CHUNK, CHUNK)],
                        osems[b],
                    )

                    @pl.when(g + NBUF < nchunk)
                    def _():
                        pltpu.make_async_copy(
                            bufs[b],
                            out_hbm.at[pl.ds(base + g * CHUNK, CHUNK)],
                            osems[b],
                        ).wait()
                        start_gather(g + NBUF, b)

            return carry

        lax.fori_loop(0, nsuper, super_body, 0)

        for b in range(NBUF):
            pltpu.make_async_copy(
                bufs[b],
                out_hbm.at[pl.ds(base, CHUNK)],
                osems[b],
            ).wait()

    return gather_kernel


def kernel(input_ids, table):
    flat = input_ids.reshape(-1).astype(jnp.int32)
    out = _gather_fn(flat.shape[0])(table, flat)
    return out.reshape(input_ids.shape + (D,))


# speedup vs baseline: 1.0094x; 1.0005x over previous
import functools

import jax
import jax.numpy as jnp
from jax import lax
from jax.experimental import pallas as pl
from jax.experimental.pallas import tpu as pltpu
from jax.experimental.pallas import tpu_sc as plsc

D = 1024
CHUNK = 8
NBUF = 15


@functools.cache
def _gather_fn(B):
    info = plsc.get_sparse_core_info()
    nw = info.num_cores * info.num_subcores
    b_per_w = B // nw
    nchunk = b_per_w // CHUNK
    nsuper = -(-nchunk // NBUF)
    mesh = plsc.VectorSubcoreMesh(core_axis_name="c", subcore_axis_name="s")

    @functools.partial(
        pl.kernel,
        mesh=mesh,
        out_type=jax.ShapeDtypeStruct((B, D), jnp.float32),
        scratch_types=[
            pltpu.VMEM((b_per_w,), jnp.int32),
            *[pltpu.VMEM((CHUNK, D), jnp.float32) for _ in range(NBUF)],
            *[pltpu.SemaphoreType.DMA for _ in range(2 * NBUF)],
        ],
    )
    def gather_kernel(table_hbm, ids_hbm, out_hbm, idx_v, *bufs_sems):
        bufs = bufs_sems[:NBUF]
        gsems = bufs_sems[NBUF : 2 * NBUF]
        osems = bufs_sems[2 * NBUF :]
        wid = lax.axis_index("s") * info.num_cores + lax.axis_index("c")
        base = wid * b_per_w

        pltpu.sync_copy(ids_hbm.at[pl.ds(base, b_per_w)], idx_v)

        def start_gather(g, b):
            pltpu.async_copy(
                table_hbm.at[idx_v.at[pl.ds(g * CHUNK, CHUNK)]], bufs[b], gsems[b]
            )

        for b in range(NBUF):
            start_gather(b, b)

        def super_body(k, carry):
            for b in range(NBUF):
                g = k * NBUF + b

                @pl.when(g < nchunk)
                def _():
                    pltpu.make_async_copy(
                        table_hbm.at[idx_v.at[pl.ds(0, CHUNK)]], bufs[b], gsems[b]
                    ).wait()
                    pltpu.async_copy(
                        bufs[b],
                        out_hbm.at[pl.ds(base + g * CHUNK, CHUNK)],
                        osems[b],
                    )

                    @pl.when(g + NBUF < nchunk)
                    def _():
                        pltpu.make_async_copy(
                            bufs[b],
                            out_hbm.at[pl.ds(base + g * CHUNK, CHUNK)],
                            osems[b],
                        ).wait()
                        start_gather(g + NBUF, b)

            return carry

        lax.fori_loop(0, nsuper, super_body, 0)

        for b in range(NBUF):
            pltpu.make_async_copy(
                bufs[b],
                out_hbm.at[pl.ds(base, CHUNK)],
                osems[b],
            ).wait()

    return gather_kernel


def kernel(input_ids, table):
    flat = input_ids.reshape(-1).astype(jnp.int32)
    out = _gather_fn(flat.shape[0])(table, flat)
    return out.reshape(input_ids.shape + (D,))


# speedup vs baseline: 1.0155x; 1.0061x over previous
import functools

import jax
import jax.numpy as jnp
from jax import lax
from jax.experimental import pallas as pl
from jax.experimental.pallas import tpu as pltpu
from jax.experimental.pallas import tpu_sc as plsc

D = 1024
CHUNK = 16
NBUF = 7


@functools.cache
def _gather_fn(B):
    info = plsc.get_sparse_core_info()
    nw = info.num_cores * info.num_subcores
    b_per_w = B // nw
    nchunk = b_per_w // CHUNK
    nsuper = -(-nchunk // NBUF)
    mesh = plsc.VectorSubcoreMesh(core_axis_name="c", subcore_axis_name="s")

    @functools.partial(
        pl.kernel,
        mesh=mesh,
        out_type=jax.ShapeDtypeStruct((B, D), jnp.float32),
        scratch_types=[
            pltpu.VMEM((b_per_w,), jnp.int32),
            *[pltpu.VMEM((CHUNK, D), jnp.float32) for _ in range(NBUF)],
            *[pltpu.SemaphoreType.DMA for _ in range(2 * NBUF)],
        ],
    )
    def gather_kernel(table_hbm, ids_hbm, out_hbm, idx_v, *bufs_sems):
        bufs = bufs_sems[:NBUF]
        gsems = bufs_sems[NBUF : 2 * NBUF]
        osems = bufs_sems[2 * NBUF :]
        wid = lax.axis_index("s") * info.num_cores + lax.axis_index("c")
        base = wid * b_per_w

        pltpu.sync_copy(ids_hbm.at[pl.ds(base, b_per_w)], idx_v)

        def start_gather(g, b):
            pltpu.async_copy(
                table_hbm.at[idx_v.at[pl.ds(g * CHUNK, CHUNK)]], bufs[b], gsems[b]
            )

        for b in range(NBUF):
            start_gather(b, b)

        def super_body(k, carry):
            for b in range(NBUF):
                g = k * NBUF + b

                @pl.when(g < nchunk)
                def _():
                    pltpu.make_async_copy(
                        table_hbm.at[idx_v.at[pl.ds(0, CHUNK)]], bufs[b], gsems[b]
                    ).wait()
                    pltpu.async_copy(
                        bufs[b],
                        out_hbm.at[pl.ds(base + g * CHUNK, CHUNK)],
                        osems[b],
                    )

                    @pl.when(g + NBUF < nchunk)
                    def _():
                        pltpu.make_async_copy(
                            bufs[b],
                            out_hbm.at[pl.ds(base + g * CHUNK, CHUNK)],
                            osems[b],
                        ).wait()
                        start_gather(g + NBUF, b)

            return carry

        lax.fori_loop(0, nsuper, super_body, 0)

        for b in range(NBUF):
            pltpu.make_async_copy(
                bufs[b],
                out_hbm.at[pl.ds(base, CHUNK)],
                osems[b],
            ).wait()

    return gather_kernel


def kernel(input_ids, table):
    flat = input_ids.reshape(-1).astype(jnp.int32)
    out = _gather_fn(flat.shape[0])(table, flat)
    return out.reshape(input_ids.shape + (D,))
